# baseline (speedup 1.0000x reference)
import functools

import jax
import jax.numpy as jnp
from jax import lax
from jax.experimental import pallas as pl
from jax.experimental.pallas import tpu as pltpu
from jax.experimental.pallas import tpu_sc as plsc

B, S, D = 4, 4096, 1024
N = B * S
NC, NS = 2, 16
NW = NC * NS
RPW = N // NW
G = 16
NG = RPW // G
NBUF = 3
L = 16


def _sc_body(x_hbm, a_hbm, m_hbm, out_hbm, xbuf, abuf, mbuf, xsem, asem, msem, osem):
    wid = lax.axis_index("s") * NC + lax.axis_index("c")
    base = wid * RPW

    def copies(g):
        slot = lax.rem(g, NBUF)
        r0 = base + g * G
        cx = pltpu.make_async_copy(x_hbm.at[pl.ds(r0, G), :], xbuf.at[slot], xsem)
        ca = pltpu.make_async_copy(a_hbm.at[pl.ds(r0, G), :], abuf.at[slot], asem)
        cm = pltpu.make_async_copy(m_hbm.at[pl.ds(r0, G), :], mbuf.at[slot], msem)
        co = pltpu.make_async_copy(xbuf.at[slot], out_hbm.at[pl.ds(r0, G), :], osem)
        return cx, ca, cm, co

    def start_in(g):
        cx, ca, cm, _ = copies(g)
        cx.start()
        ca.start()
        cm.start()

    start_in(0)

    def step(g, carry):
        cx, ca, cm, co = copies(g)

        @pl.when(g + 1 - NBUF >= 0)
        def _():
            copies(g + 1 - NBUF)[3].wait()

        @pl.when(g + 1 < NG)
        def _():
            start_in(g + 1)

        cx.wait()
        ca.wait()
        cm.wait()

        slot = lax.rem(g, NBUF)

        def crow(r, c):
            mv = mbuf[slot, r, :]
            for k in range(D // L):
                av = abuf[slot, r, pl.ds(k * L, L)]
                xv = xbuf[slot, r, pl.ds(k * L, L)]
                xbuf[slot, r, pl.ds(k * L, L)] = xv + av * mv
            return c

        lax.fori_loop(0, G, crow, 0)
        co.start()
        return carry

    lax.fori_loop(0, NG, step, 0)
    copies(NG - 2)[3].wait()
    copies(NG - 1)[3].wait()


_sc_kernel = functools.partial(
    pl.kernel,
    mesh=plsc.VectorSubcoreMesh(core_axis_name="c", subcore_axis_name="s"),
    out_type=jax.ShapeDtypeStruct((N, D), jnp.float32),
    scratch_types=[
        pltpu.VMEM((NBUF, G, D), jnp.float32),
        pltpu.VMEM((NBUF, G, D), jnp.float32),
        pltpu.VMEM((NBUF, G, L), jnp.float32),
        pltpu.SemaphoreType.DMA,
        pltpu.SemaphoreType.DMA,
        pltpu.SemaphoreType.DMA,
        pltpu.SemaphoreType.DMA,
    ],
)(_sc_body)


def kernel(x, attack, attack_mask):
    xr = x.reshape(N, D)
    ar = attack.reshape(N, D)
    m16 = jnp.broadcast_to(
        attack_mask.reshape(N, 1), (N, L)
    ).astype(jnp.float32)
    out = _sc_kernel(xr, ar, m16)
    return out.reshape(B, S, D)


# speedup vs baseline: 2.1731x; 2.1731x over previous
import functools

import jax
import jax.numpy as jnp
from jax import lax
from jax.experimental import pallas as pl
from jax.experimental.pallas import tpu as pltpu
from jax.experimental.pallas import tpu_sc as plsc

B, S, D = 4, 4096, 1024
N = B * S
NC, NS = 2, 16
NW = NC * NS
RPW = N // NW
G = 16
NG = RPW // G
NBUF = 3
L = 16


def _sc_body(x_hbm, a_hbm, m_hbm, out_hbm, xbuf, abuf, mbuf, xsem, asem, msem, osem):
    wid = lax.axis_index("s") * NC + lax.axis_index("c")
    base = wid * RPW

    def copies(g):
        slot = lax.rem(g, NBUF)
        r0 = base + g * G
        cx = pltpu.make_async_copy(x_hbm.at[pl.ds(r0, G), :], xbuf.at[slot], xsem)
        ca = pltpu.make_async_copy(a_hbm.at[pl.ds(r0, G), :], abuf.at[slot], asem)
        cm = pltpu.make_async_copy(m_hbm.at[pl.ds(r0, G), :], mbuf.at[slot], msem)
        co = pltpu.make_async_copy(xbuf.at[slot], out_hbm.at[pl.ds(r0, G), :], osem)
        return cx, ca, cm, co

    def start_in(g):
        cx, ca, cm, _ = copies(g)
        cx.start()
        ca.start()
        cm.start()

    start_in(0)

    def step(g, carry):
        cx, ca, cm, co = copies(g)

        @pl.when(g + 1 - NBUF >= 0)
        def _():
            copies(g + 1 - NBUF)[3].wait()

        @pl.when(g + 1 < NG)
        def _():
            start_in(g + 1)

        cx.wait()
        ca.wait()
        cm.wait()

        slot = lax.rem(g, NBUF)

        @plsc.parallel_loop(0, G)
        def _rows(r):
            mv = mbuf[slot, r, :]
            for k in range(D // L):
                av = abuf[slot, r, pl.ds(k * L, L)]
                plsc.addupdate(xbuf.at[slot, r, pl.ds(k * L, L)], av * mv)

        co.start()
        return carry

    lax.fori_loop(0, NG, step, 0)
    copies(NG - 2)[3].wait()
    copies(NG - 1)[3].wait()


_sc_kernel = functools.partial(
    pl.kernel,
    mesh=plsc.VectorSubcoreMesh(core_axis_name="c", subcore_axis_name="s"),
    out_type=jax.ShapeDtypeStruct((N, D), jnp.float32),
    scratch_types=[
        pltpu.VMEM((NBUF, G, D), jnp.float32),
        pltpu.VMEM((NBUF, G, D), jnp.float32),
        pltpu.VMEM((NBUF, G, L), jnp.float32),
        pltpu.SemaphoreType.DMA,
        pltpu.SemaphoreType.DMA,
        pltpu.SemaphoreType.DMA,
        pltpu.SemaphoreType.DMA,
    ],
)(_sc_body)


def kernel(x, attack, attack_mask):
    xr = x.reshape(N, D)
    ar = attack.reshape(N, D)
    m16 = jnp.broadcast_to(
        attack_mask.reshape(N, 1), (N, L)
    ).astype(jnp.float32)
    out = _sc_kernel(xr, ar, m16)
    return out.reshape(B, S, D)
